# baseline (speedup 1.0000x reference)
import functools

import jax
import jax.numpy as jnp
import numpy as np
from jax import lax
from jax.experimental import pallas as pl
from jax.experimental.pallas import tpu as pltpu
from jax.experimental.pallas import tpu_sc as plsc

VOCAB = 100000
D = 64
NS = 26
B = 4096
NF = NS + 1

NW = 32
ROWS = B * NS
RPW = ROWS // NW
CHUNK = 128
NCH = RPW // CHUNK


def _sc_gather(tables_flat, idx_flat):
  mesh = plsc.VectorSubcoreMesh(core_axis_name="c", subcore_axis_name="s")

  @functools.partial(
      pl.kernel,
      mesh=mesh,
      out_type=jax.ShapeDtypeStruct((ROWS, D), jnp.float32),
      compiler_params=pltpu.CompilerParams(use_tc_tiling_on_sc=False),
      scratch_types=[
          pltpu.VMEM((RPW,), jnp.int32),
          pltpu.VMEM((CHUNK, D), jnp.float32),
          pltpu.SemaphoreType.DMA,
      ],
  )
  def gather_k(tab_hbm, idx_hbm, out_hbm, idx_v, rows_v, sem):
    wid = lax.axis_index("s") * 2 + lax.axis_index("c")
    pltpu.sync_copy(idx_hbm.at[pl.ds(wid * RPW, RPW)], idx_v)
    for c in range(NCH):
      pltpu.async_copy(
          tab_hbm.at[idx_v.at[pl.ds(c * CHUNK, CHUNK)]], rows_v, sem).wait()
      pltpu.sync_copy(rows_v, out_hbm.at[pl.ds(wid * RPW + c * CHUNK, CHUNK)])

  return gather_k(tables_flat, idx_flat)


BT = 256
GRID = B // BT


def _tc_body(dense_ref, emb_ref, bW0, bb0, bW1, bb1, bW2, bb2,
             tW0d, Wg3, tb0, tW1, tb1, tW2, tb2, out_ref):
  relu = lambda v: jnp.maximum(v, 0.0)
  x = dense_ref[...]
  h = relu(jnp.dot(x, bW0[...], preferred_element_type=jnp.float32) + bb0[...])
  h = relu(jnp.dot(h, bW1[...], preferred_element_type=jnp.float32) + bb1[...])
  dout = relu(jnp.dot(h, bW2[...], preferred_element_type=jnp.float32) + bb2[...])

  emb3 = emb_ref[...]
  f3 = jnp.concatenate([dout.reshape(BT, 1, D), emb3], axis=1)
  g = lax.dot_general(f3, f3, (((2,), (2,)), ((0,), (0,))),
                      preferred_element_type=jnp.float32)

  z = jnp.dot(dout, tW0d[...], preferred_element_type=jnp.float32) + tb0[...]
  for n in range(NF):
    z = z + jnp.dot(g[:, n, :], Wg3[n], preferred_element_type=jnp.float32)
  y = relu(z)
  y = relu(jnp.dot(y, tW1[...], preferred_element_type=jnp.float32) + tb1[...])
  y = relu(jnp.dot(y, tW2[...], preferred_element_type=jnp.float32) + tb2[...])
  out_ref[...] = y


def _tc_call(dense, emb3, bW0, bb0, bW1, bb1, bW2, bb2,
             tW0d, Wg3, tb0, tW1, tb1, tW2, tb2):
  full2 = lambda shape: pl.BlockSpec(shape, lambda i: (0, 0))
  return pl.pallas_call(
      _tc_body,
      grid=(GRID,),
      in_specs=[
          pl.BlockSpec((BT, 13), lambda i: (i, 0)),
          pl.BlockSpec((BT, NS, D), lambda i: (i, 0, 0)),
          full2((13, 512)), full2((1, 512)),
          full2((512, 256)), full2((1, 256)),
          full2((256, 64)), full2((1, 64)),
          full2((64, 512)),
          pl.BlockSpec((NF, NF, 512), lambda i: (0, 0, 0)),
          full2((1, 512)),
          full2((512, 256)), full2((1, 256)),
          full2((256, 1)), full2((1, 1)),
      ],
      out_specs=pl.BlockSpec((BT, 1), lambda i: (i, 0)),
      out_shape=jax.ShapeDtypeStruct((B, 1), jnp.float32),
  )(dense, emb3, bW0, bb0, bW1, bb1, bW2, bb2,
    tW0d, Wg3, tb0, tW1, tb1, tW2, tb2)


def kernel(dense, sparse, tables, bW0, bb0, bW1, bb1, bW2, bb2,
           tW0, tb0, tW1, tb1, tW2, tb2):
  sparse = sparse.astype(jnp.int32)
  offs = (jnp.arange(NS, dtype=jnp.int32) * VOCAB)[None, :]
  idx_flat = (sparse + offs).reshape(ROWS)
  tables_flat = tables.reshape(NS * VOCAB, D)
  emb = _sc_gather(tables_flat, idx_flat)
  emb3 = emb.reshape(B, NS, D)

  tri0, tri1 = np.triu_indices(NF, k=1)
  tW0d = tW0[:D]
  Wg3 = jnp.zeros((NF, NF, 512), jnp.float32).at[tri0, tri1].set(tW0[D:])

  out = _tc_call(
      dense, emb3, bW0, bb0.reshape(1, -1), bW1, bb1.reshape(1, -1),
      bW2, bb2.reshape(1, -1), tW0d, Wg3, tb0.reshape(1, -1),
      tW1, tb1.reshape(1, -1), tW2, tb2.reshape(1, -1))
  return out[:, 0]
